# baseline (speedup 1.0000x reference)
import functools

import jax
import jax.numpy as jnp
from jax import lax
from jax.experimental import pallas as pl
from jax.experimental.pallas import tpu as pltpu
from jax.experimental.pallas import tpu_sc as plsc

HIDDEN = 128
BATCH = 16384
NUM_CORES = 2
NUM_SUBCORES = 16
NW = NUM_CORES * NUM_SUBCORES
B_PER_W = BATCH // NW
CHUNK = 128
N_CHUNKS = B_PER_W // CHUNK
LANES = 16


def _make_kernel():
    mesh = plsc.VectorSubcoreMesh(core_axis_name="c", subcore_axis_name="s")

    @functools.partial(
        pl.kernel,
        mesh=mesh,
        out_type=jax.ShapeDtypeStruct((BATCH, HIDDEN), jnp.float32),
        scratch_types=[
            pltpu.VMEM((N_CHUNKS, CHUNK), jnp.int32),
            pltpu.VMEM((B_PER_W, HIDDEN), jnp.float32),
        ]
        + [pltpu.SemaphoreType.DMA] * (N_CHUNKS + 1),
    )
    def k(x_hbm, table_hbm, out_hbm, idx_v, rows_v, *sems):
        gather_sems, out_sem = sems[:N_CHUNKS], sems[N_CHUNKS]
        wid = lax.axis_index("s") * NUM_CORES + lax.axis_index("c")
        base = wid * B_PER_W
        pltpu.sync_copy(x_hbm.at[wid], idx_v)
        gathers = []
        for j in range(N_CHUNKS):
            for i in range(CHUNK // LANES):
                sl = pl.ds(i * LANES, LANES)
                idx_v[j, sl] = idx_v[j, sl] - 1
            gathers.append(
                pltpu.async_copy(
                    table_hbm.at[idx_v.at[j]],
                    rows_v.at[pl.ds(j * CHUNK, CHUNK)],
                    gather_sems[j],
                )
            )
        outs = []
        for j in range(N_CHUNKS):
            gathers[j].wait()
            outs.append(
                pltpu.async_copy(
                    rows_v.at[pl.ds(j * CHUNK, CHUNK)],
                    out_hbm.at[pl.ds(base + j * CHUNK, CHUNK)],
                    out_sem,
                )
            )
        for c in outs:
            c.wait()

    return k


_sc_lookup = _make_kernel()


def kernel(x, table):
    idx3 = x.reshape(NW, N_CHUNKS, CHUNK)
    return _sc_lookup(idx3, table)


# speedup vs baseline: 1.3509x; 1.3509x over previous
import functools

import jax
import jax.numpy as jnp
from jax import lax
from jax.experimental import pallas as pl
from jax.experimental.pallas import tpu as pltpu
from jax.experimental.pallas import tpu_sc as plsc

HIDDEN = 128
BATCH = 16384
NUM_CORES = 2
NUM_SUBCORES = 16
NW = NUM_CORES * NUM_SUBCORES
B_PER_W = BATCH // NW
CHUNK = 128
N_CHUNKS = B_PER_W // CHUNK
LANES = 16


def _make_kernel():
    mesh = plsc.VectorSubcoreMesh(core_axis_name="c", subcore_axis_name="s")

    @functools.partial(
        pl.kernel,
        mesh=mesh,
        out_type=jax.ShapeDtypeStruct((BATCH, HIDDEN), jnp.float32),
        scratch_types=[
            pltpu.VMEM((N_CHUNKS, CHUNK), jnp.int32),
            pltpu.VMEM((B_PER_W, HIDDEN), jnp.float32),
            pltpu.VMEM_SHARED((230, HIDDEN), jnp.float32),
        ]
        + [pltpu.SemaphoreType.DMA] * (N_CHUNKS + 1),
    )
    def k(x_hbm, table_hbm, out_hbm, idx_v, rows_v, table_sh, *sems):
        gather_sems, out_sem = sems[:N_CHUNKS], sems[N_CHUNKS]
        sid = lax.axis_index("s")
        wid = sid * NUM_CORES + lax.axis_index("c")
        base = wid * B_PER_W
        @pl.when(sid == 0)
        def _():
            pltpu.sync_copy(table_hbm, table_sh)

        pltpu.sync_copy(x_hbm.at[wid], idx_v)
        plsc.subcore_barrier()
        gathers = []
        for j in range(N_CHUNKS):
            for i in range(CHUNK // LANES):
                sl = pl.ds(i * LANES, LANES)
                idx_v[j, sl] = idx_v[j, sl] - 1
            gathers.append(
                pltpu.async_copy(
                    table_sh.at[idx_v.at[j]],
                    rows_v.at[pl.ds(j * CHUNK, CHUNK)],
                    gather_sems[j],
                )
            )
        outs = []
        for j in range(N_CHUNKS):
            gathers[j].wait()
            outs.append(
                pltpu.async_copy(
                    rows_v.at[pl.ds(j * CHUNK, CHUNK)],
                    out_hbm.at[pl.ds(base + j * CHUNK, CHUNK)],
                    out_sem,
                )
            )
        for c in outs:
            c.wait()

    return k


_sc_lookup = _make_kernel()


def kernel(x, table):
    idx3 = x.reshape(NW, N_CHUNKS, CHUNK)
    return _sc_lookup(idx3, table)


# speedup vs baseline: 1.3811x; 1.0223x over previous
import functools

import jax
import jax.numpy as jnp
from jax import lax
from jax.experimental import pallas as pl
from jax.experimental.pallas import tpu as pltpu
from jax.experimental.pallas import tpu_sc as plsc

HIDDEN = 128
BATCH = 16384
NUM_CORES = 2
NUM_SUBCORES = 16
NW = NUM_CORES * NUM_SUBCORES
B_PER_W = BATCH // NW
CHUNK = 128
N_CHUNKS = B_PER_W // CHUNK
LANES = 16


def _make_kernel():
    mesh = plsc.VectorSubcoreMesh(core_axis_name="c", subcore_axis_name="s")

    @functools.partial(
        pl.kernel,
        mesh=mesh,
        out_type=jax.ShapeDtypeStruct((BATCH, HIDDEN), jnp.float32),
        scratch_types=[
            pltpu.VMEM((N_CHUNKS, CHUNK), jnp.int32),
            pltpu.VMEM((B_PER_W, HIDDEN), jnp.float32),
            pltpu.VMEM_SHARED((230, HIDDEN), jnp.float32),
        ]
        + [pltpu.SemaphoreType.DMA] * (N_CHUNKS + 1),
    )
    def k(x_hbm, table_hbm, out_hbm, idx_v, rows_v, table_sh, *sems):
        gather_sems, out_sem = sems[:N_CHUNKS], sems[N_CHUNKS]
        sid = lax.axis_index("s")
        wid = sid * NUM_CORES + lax.axis_index("c")
        base = wid * B_PER_W
        idx_cp = pltpu.async_copy(x_hbm.at[wid], idx_v, out_sem)

        @pl.when(sid == 0)
        def _():
            pltpu.sync_copy(table_hbm, table_sh)

        idx_cp.wait()
        for j in range(N_CHUNKS):
            for i in range(CHUNK // LANES):
                sl = pl.ds(i * LANES, LANES)
                idx_v[j, sl] = idx_v[j, sl] - 1
        plsc.subcore_barrier()
        gathers = []
        for j in range(N_CHUNKS):
            gathers.append(
                pltpu.async_copy(
                    table_sh.at[idx_v.at[j]],
                    rows_v.at[pl.ds(j * CHUNK, CHUNK)],
                    gather_sems[j],
                )
            )
        outs = []
        for j in range(N_CHUNKS):
            gathers[j].wait()
            outs.append(
                pltpu.async_copy(
                    rows_v.at[pl.ds(j * CHUNK, CHUNK)],
                    out_hbm.at[pl.ds(base + j * CHUNK, CHUNK)],
                    out_sem,
                )
            )
        for c in outs:
            c.wait()

    return k


_sc_lookup = _make_kernel()


def kernel(x, table):
    idx3 = x.reshape(NW, N_CHUNKS, CHUNK)
    return _sc_lookup(idx3, table)


# speedup vs baseline: 1.3901x; 1.0065x over previous
import functools

import jax
import jax.numpy as jnp
from jax import lax
from jax.experimental import pallas as pl
from jax.experimental.pallas import tpu as pltpu
from jax.experimental.pallas import tpu_sc as plsc

HIDDEN = 128
BATCH = 16384
NUM_CORES = 2
NUM_SUBCORES = 16
NW = NUM_CORES * NUM_SUBCORES
B_PER_W = BATCH // NW
CHUNK = 64
N_CHUNKS = B_PER_W // CHUNK
LANES = 16


def _make_kernel():
    mesh = plsc.VectorSubcoreMesh(core_axis_name="c", subcore_axis_name="s")

    @functools.partial(
        pl.kernel,
        mesh=mesh,
        out_type=jax.ShapeDtypeStruct((BATCH, HIDDEN), jnp.float32),
        scratch_types=[
            pltpu.VMEM((N_CHUNKS, CHUNK), jnp.int32),
            pltpu.VMEM((B_PER_W, HIDDEN), jnp.float32),
            pltpu.VMEM_SHARED((230, HIDDEN), jnp.float32),
        ]
        + [pltpu.SemaphoreType.DMA] * (N_CHUNKS + 1),
    )
    def k(x_hbm, table_hbm, out_hbm, idx_v, rows_v, table_sh, *sems):
        gather_sems, out_sem = sems[:N_CHUNKS], sems[N_CHUNKS]
        sid = lax.axis_index("s")
        wid = sid * NUM_CORES + lax.axis_index("c")
        base = wid * B_PER_W
        idx_cp = pltpu.async_copy(x_hbm.at[wid], idx_v, out_sem)

        @pl.when(sid == 0)
        def _():
            pltpu.sync_copy(table_hbm, table_sh)

        idx_cp.wait()
        for j in range(N_CHUNKS):
            for i in range(CHUNK // LANES):
                sl = pl.ds(i * LANES, LANES)
                idx_v[j, sl] = idx_v[j, sl] - 1
        plsc.subcore_barrier()
        gathers = []
        for j in range(N_CHUNKS):
            gathers.append(
                pltpu.async_copy(
                    table_sh.at[idx_v.at[j]],
                    rows_v.at[pl.ds(j * CHUNK, CHUNK)],
                    gather_sems[j],
                )
            )
        outs = []
        for j in range(N_CHUNKS):
            gathers[j].wait()
            outs.append(
                pltpu.async_copy(
                    rows_v.at[pl.ds(j * CHUNK, CHUNK)],
                    out_hbm.at[pl.ds(base + j * CHUNK, CHUNK)],
                    out_sem,
                )
            )
        for c in outs:
            c.wait()

    return k


_sc_lookup = _make_kernel()


def kernel(x, table):
    idx3 = x.reshape(NW, N_CHUNKS, CHUNK)
    return _sc_lookup(idx3, table)
